# Initial kernel scaffold; baseline (speedup 1.0000x reference)
import functools

import jax
import jax.numpy as jnp
from jax import lax
from jax.experimental import pallas as pl
from jax.experimental.pallas import tpu as pltpu
from jax.experimental.pallas import tpu_sc as plsc

N = 10000
E = 320000
ET = 22
ATP = 104
NPAD = 10240
EPAD = 327680
EPT = EPAD // 32
CHUNK = 1024
ROWS_PT = NPAD // 16
F32 = jnp.float32
I32 = jnp.int32


def _stage_a(atom_ref, ee_ref, w1r_ref, b1r_ref, root1_ref, b1v_ref,
             w2r_ref, b2r_ref, a1_ref, r1_ref, w2t_ref):
    at = atom_ref[...]
    for t in range(ET):
        w = b1r_ref[...]
        w2 = b2r_ref[...]
        for e in range(16):
            s = ee_ref[t:t + 1, e:e + 1]
            w = w + s * w1r_ref[e]
            w2 = w2 + s * w2r_ref[e]
        a1_ref[t] = jnp.dot(at, w, preferred_element_type=F32)
        w2t_ref[t] = w2
    r1_ref[...] = jnp.dot(at, root1_ref[...], preferred_element_type=F32) + b1v_ref[...]


def _run_stage_a(atom_pad, edge_emb, w1r, b1r, root1, bias1, w2r, b2r):
    return pl.pallas_call(
        _stage_a,
        out_shape=[
            jax.ShapeDtypeStruct((ET, ATP, 16), F32),
            jax.ShapeDtypeStruct((ATP, 16), F32),
            jax.ShapeDtypeStruct((ET, 16, 16), F32),
        ],
    )(atom_pad, edge_emb, w1r, b1r, root1, bias1.reshape(1, 16), w2r, b2r)


def _stage_b(p0_ref, p1_ref, w2t_ref, root2_ref, b2v_ref, y2_ref, r2_ref):
    h = jnp.maximum(p0_ref[...] + p1_ref[...], 0.0)
    for t in range(ET):
        y2_ref[t] = jnp.dot(h, w2t_ref[t], preferred_element_type=F32)
    r2_ref[...] = jnp.dot(h, root2_ref[...], preferred_element_type=F32) + b2v_ref[...]


def _run_stage_b(p0, p1, w2t, root2, bias2):
    nb = 2000
    grid = N // nb
    return pl.pallas_call(
        _stage_b,
        grid=(grid,),
        in_specs=[
            pl.BlockSpec((nb, 16), lambda j: (j, 0)),
            pl.BlockSpec((nb, 16), lambda j: (j, 0)),
            pl.BlockSpec((ET, 16, 16), lambda j: (0, 0, 0)),
            pl.BlockSpec((16, 16), lambda j: (0, 0)),
            pl.BlockSpec((1, 16), lambda j: (0, 0)),
        ],
        out_specs=[
            pl.BlockSpec((ET, nb, 16), lambda j: (0, j, 0)),
            pl.BlockSpec((nb, 16), lambda j: (j, 0)),
        ],
        out_shape=[
            jax.ShapeDtypeStruct((ET, N, 16), F32),
            jax.ShapeDtypeStruct((N, 16), F32),
        ],
    )(p0, p1, w2t, root2, bias2.reshape(1, 16))


def _stage_c(q0_ref, q1_ref, out_ref):
    out_ref[...] = q0_ref[...] + q1_ref[...]


def _run_stage_c(q0, q1):
    return pl.pallas_call(
        _stage_c,
        out_shape=jax.ShapeDtypeStruct((N, 16), F32),
    )(q0, q1)


_MESH = plsc.VectorSubcoreMesh(core_axis_name="c", subcore_axis_name="s",
                               num_cores=2, num_subcores=16)


def _sc_body_l1(src_h, dst_h, et_h, x_h, a1_h, r1_h, zer_h, out_h,
                acc, srcb, etb, dst2d, gidx2d, msgb, xinit, initb, xtab, sem):
    c = lax.axis_index("c")
    s = lax.axis_index("s")
    rbase = s * ROWS_PT
    pltpu.sync_copy(x_h, xtab)

    @pl.when(c == 0)
    def _():
        for j in range(5):
            pltpu.sync_copy(x_h.at[pl.ds(rbase + j * 128, 128)], xinit.at[j])
        for j in range(5):
            pltpu.async_copy(r1_h.at[xinit.at[j]],
                             initb.at[pl.ds(j * 128, 128)], sem).wait()
        pltpu.sync_copy(initb, acc.at[pl.ds(rbase, ROWS_PT)])

    @pl.when(c == 1)
    def _():
        pltpu.sync_copy(zer_h.at[pl.ds(rbase, ROWS_PT)],
                        acc.at[pl.ds(rbase, ROWS_PT)])

    plsc.subcore_barrier()

    eb = (c * 16 + s) * EPT

    def chunk(k, carry):
        off = eb + k * CHUNK
        pltpu.sync_copy(src_h.at[pl.ds(off, CHUNK)], srcb)
        pltpu.sync_copy(et_h.at[pl.ds(off, CHUNK)], etb)
        for j in range(8):
            pltpu.sync_copy(dst_h.at[pl.ds(off + j * 128, 128)], dst2d.at[j])
        for i in range(CHUNK // 16):
            sv = srcb[pl.ds(i * 16, 16)]
            tv = etb[pl.ds(i * 16, 16)]
            xv = plsc.load_gather(xtab, [sv])
            gidx2d[i // 8, pl.ds((i % 8) * 16, 16)] = tv * ATP + xv
        cps = [pltpu.async_copy(a1_h.at[gidx2d.at[j]],
                                msgb.at[pl.ds(j * 128, 128)], sem)
               for j in range(8)]
        for cp in cps:
            cp.wait()
        for j in range(8):
            pltpu.sync_copy(msgb.at[pl.ds(j * 128, 128)],
                            acc.at[dst2d.at[j]], add=True)
        return carry

    lax.fori_loop(0, EPT // CHUNK, chunk, 0)
    plsc.subcore_barrier()
    pltpu.sync_copy(acc.at[pl.ds(rbase, ROWS_PT)],
                    out_h.at[c, pl.ds(rbase, ROWS_PT)])


def _sc_body_l2(src_h, dst_h, et_h, y2_h, r2_h, zer_h, out_h,
                acc, srcb, etb, dst2d, gidx2d, msgb, sem):
    c = lax.axis_index("c")
    s = lax.axis_index("s")
    rbase = s * ROWS_PT

    @pl.when(c == 0)
    def _():
        pltpu.sync_copy(r2_h.at[pl.ds(rbase, ROWS_PT)],
                        acc.at[pl.ds(rbase, ROWS_PT)])

    @pl.when(c == 1)
    def _():
        pltpu.sync_copy(zer_h.at[pl.ds(rbase, ROWS_PT)],
                        acc.at[pl.ds(rbase, ROWS_PT)])

    plsc.subcore_barrier()

    eb = (c * 16 + s) * EPT

    def chunk(k, carry):
        off = eb + k * CHUNK
        pltpu.sync_copy(src_h.at[pl.ds(off, CHUNK)], srcb)
        pltpu.sync_copy(et_h.at[pl.ds(off, CHUNK)], etb)
        for j in range(8):
            pltpu.sync_copy(dst_h.at[pl.ds(off + j * 128, 128)], dst2d.at[j])
        for i in range(CHUNK // 16):
            sv = srcb[pl.ds(i * 16, 16)]
            tv = etb[pl.ds(i * 16, 16)]
            gidx2d[i // 8, pl.ds((i % 8) * 16, 16)] = tv * N + sv
        cps = [pltpu.async_copy(y2_h.at[gidx2d.at[j]],
                                msgb.at[pl.ds(j * 128, 128)], sem)
               for j in range(8)]
        for cp in cps:
            cp.wait()
        for j in range(8):
            pltpu.sync_copy(msgb.at[pl.ds(j * 128, 128)],
                            acc.at[dst2d.at[j]], add=True)
        return carry

    lax.fori_loop(0, EPT // CHUNK, chunk, 0)
    plsc.subcore_barrier()
    pltpu.sync_copy(acc.at[pl.ds(rbase, ROWS_PT)],
                    out_h.at[c, pl.ds(rbase, ROWS_PT)])


def _sc_common_scratch():
    return [
        pltpu.VMEM((CHUNK,), I32),
        pltpu.VMEM((CHUNK,), I32),
        pltpu.VMEM((8, 128), I32),
        pltpu.VMEM((8, 128), I32),
        pltpu.VMEM((CHUNK, 16), F32),
    ]


def _run_sc_l1(src1, dstp, etp, xpad, a1f, r1, zer):
    k = pl.kernel(
        _sc_body_l1,
        out_type=jax.ShapeDtypeStruct((2, NPAD, 16), F32),
        mesh=_MESH,
        scratch_types=[pltpu.VMEM_SHARED((NPAD, 16), F32)]
        + _sc_common_scratch()
        + [
            pltpu.VMEM((5, 128), I32),
            pltpu.VMEM((ROWS_PT, 16), F32),
            pltpu.VMEM((NPAD,), I32),
            pltpu.SemaphoreType.DMA,
        ],
    )
    return k(src1, dstp, etp, xpad, a1f, r1, zer)


def _run_sc_l2(src2, dstp, etp, y2f, r2pad, zer):
    k = pl.kernel(
        _sc_body_l2,
        out_type=jax.ShapeDtypeStruct((2, NPAD, 16), F32),
        mesh=_MESH,
        scratch_types=[pltpu.VMEM_SHARED((NPAD, 16), F32)]
        + _sc_common_scratch()
        + [pltpu.SemaphoreType.DMA],
    )
    return k(src2, dstp, etp, y2f, r2pad, zer)


def kernel(x, edge_index, edge_attr, atom_emb, edge_emb,
           lin1_W, lin1_b, root1, bias1,
           lin2_W, lin2_b, root2, bias2):
    xs = x[:, 0].astype(I32)
    src = edge_index[0].astype(I32)
    dst = edge_index[1].astype(I32)
    et = edge_attr[:, 0].astype(I32)

    xpad = jnp.concatenate([xs, jnp.full((NPAD - N,), 101, I32)])
    pe = EPAD - E
    src1 = jnp.concatenate([src, jnp.full((pe,), N, I32)])
    src2 = jnp.concatenate([src, jnp.full((pe,), ET * N, I32)])
    dstp = jnp.concatenate([dst, jnp.zeros((pe,), I32)])
    etp = jnp.concatenate([et, jnp.zeros((pe,), I32)])

    atom_pad = jnp.concatenate([atom_emb, jnp.zeros((ATP - 101, 16), F32)])
    w1r = lin1_W.reshape(16, 16, 16)
    b1r = lin1_b.reshape(16, 16)
    w2r = lin2_W.reshape(16, 16, 16)
    b2r = lin2_b.reshape(16, 16)

    a1, r1, w2t = _run_stage_a(atom_pad, edge_emb, w1r, b1r, root1, bias1,
                               w2r, b2r)
    a1f = a1.reshape(ET * ATP, 16)

    zer = jnp.zeros((NPAD, 16), F32)
    p = _run_sc_l1(src1, dstp, etp, xpad, a1f, r1, zer)

    y2, r2 = _run_stage_b(p[0, :N], p[1, :N], w2t, root2, bias2)
    y2f = jnp.concatenate([y2.reshape(ET * N, 16), jnp.zeros((32, 16), F32)])
    r2pad = jnp.concatenate([r2, jnp.zeros((NPAD - N, 16), F32)])

    q = _run_sc_l2(src2, dstp, etp, y2f, r2pad, zer)

    return _run_stage_c(q[0, :N], q[1, :N])


# speedup vs baseline: 11.4347x; 11.4347x over previous
import functools

import jax
import jax.numpy as jnp
from jax import lax
from jax.experimental import pallas as pl
from jax.experimental.pallas import tpu as pltpu
from jax.experimental.pallas import tpu_sc as plsc

N = 10000
E = 320000
ET = 22
ATP = 104
NPAD = 10240
EPAD = 327680
EPT = EPAD // 32
CHUNK = 2048
NCH = EPT // CHUNK
ROWS_PT = NPAD // 16
F32 = jnp.float32
I32 = jnp.int32


def _stage_a(atom_ref, ee_ref, w1r_ref, b1r_ref, root1_ref, b1v_ref,
             w2r_ref, b2r_ref, a1_ref, r1_ref, w2cat_ref):
    at = atom_ref[...]
    w1cat = jnp.concatenate(
        [b1r_ref[...]
         + sum(ee_ref[t:t + 1, e:e + 1] * w1r_ref[e] for e in range(16))
         for t in range(ET)], axis=1)
    w2cat = jnp.concatenate(
        [b2r_ref[...]
         + sum(ee_ref[t:t + 1, e:e + 1] * w2r_ref[e] for e in range(16))
         for t in range(ET)], axis=1)
    w2cat_ref[...] = w2cat
    a1cat = jnp.dot(at, w1cat, preferred_element_type=F32)
    for t in range(ET):
        a1_ref[t] = a1cat[:, t * 16:(t + 1) * 16]
    r1_ref[...] = jnp.dot(at, root1_ref[...],
                          preferred_element_type=F32) + b1v_ref[...]


def _run_stage_a(atom_pad, edge_emb, w1r, b1r, root1, bias1, w2r, b2r):
    return pl.pallas_call(
        _stage_a,
        out_shape=[
            jax.ShapeDtypeStruct((ET, ATP, 16), F32),
            jax.ShapeDtypeStruct((ATP, 16), F32),
            jax.ShapeDtypeStruct((16, ET * 16), F32),
        ],
    )(atom_pad, edge_emb, w1r, b1r, root1, bias1.reshape(1, 16), w2r, b2r)


def _stage_b(p0_ref, p1_ref, w2cat_ref, root2_ref, b2v_ref, y2_ref, r2_ref):
    h = jnp.maximum(p0_ref[...] + p1_ref[...], 0.0)
    ycat = jnp.dot(h, w2cat_ref[...], preferred_element_type=F32)
    for t in range(ET):
        y2_ref[t] = ycat[:, t * 16:(t + 1) * 16]
    r2_ref[...] = jnp.dot(h, root2_ref[...],
                          preferred_element_type=F32) + b2v_ref[...]


def _run_stage_b(p0, p1, w2cat, root2, bias2):
    nb = 2000
    return pl.pallas_call(
        _stage_b,
        grid=(N // nb,),
        in_specs=[
            pl.BlockSpec((nb, 16), lambda j: (j, 0)),
            pl.BlockSpec((nb, 16), lambda j: (j, 0)),
            pl.BlockSpec((16, ET * 16), lambda j: (0, 0)),
            pl.BlockSpec((16, 16), lambda j: (0, 0)),
            pl.BlockSpec((1, 16), lambda j: (0, 0)),
        ],
        out_specs=[
            pl.BlockSpec((ET, nb, 16), lambda j: (0, j, 0)),
            pl.BlockSpec((nb, 16), lambda j: (j, 0)),
        ],
        out_shape=[
            jax.ShapeDtypeStruct((ET, N, 16), F32),
            jax.ShapeDtypeStruct((N, 16), F32),
        ],
    )(p0, p1, w2cat, root2, bias2.reshape(1, 16))


def _stage_c(q0_ref, q1_ref, out_ref):
    out_ref[...] = q0_ref[...] + q1_ref[...]


def _run_stage_c(q0, q1):
    return pl.pallas_call(
        _stage_c,
        out_shape=jax.ShapeDtypeStruct((N, 16), F32),
    )(q0, q1)


@functools.cache
def _mesh():
    return plsc.VectorSubcoreMesh(core_axis_name="c", subcore_axis_name="s",
                                  num_cores=2, num_subcores=16)


def _edge_pipeline(table_h, src_h, dst_h, et_h, acc, eb, bufs, sems,
                   gidx_of, x1d_h=None):
    srcb, etb, dstb, gidxb, msgb, xsrcb = bufs
    semL, semX, semG, semD = sems
    has_x = x1d_h is not None

    def fire_lin(k, b):
        off = eb + k * CHUNK
        return [pltpu.async_copy(src_h.at[pl.ds(off, CHUNK)], srcb[b], semL),
                pltpu.async_copy(et_h.at[pl.ds(off, CHUNK)], etb[b], semL)]

    def fire_dst(k, b):
        off = eb + k * CHUNK
        return pltpu.async_copy(dst_h.at[pl.ds(off, CHUNK)], dstb[b], semD)

    def fire_xg(b):
        return pltpu.async_copy(x1d_h.at[srcb[b]], xsrcb[b], semX)

    def compute(b):
        for i in range(CHUNK // 16):
            sl = pl.ds(i * 16, 16)
            tv = etb[b][sl]
            sv = xsrcb[b][sl] if has_x else srcb[b][sl]
            gidxb[b][sl] = gidx_of(tv, sv)

    lin = {0: fire_lin(0, 0)}
    xg = {}
    g = {}
    d = {}
    if has_x:
        for cp in lin[0]:
            cp.wait()
        xg[0] = fire_xg(0)
        lin[1] = fire_lin(1, 1)
    for k in range(NCH):
        b = k & 1
        if has_x:
            xg[k].wait()
        else:
            for cp in lin[k]:
                cp.wait()
        compute(b)
        if k > 0:
            g[k - 1].wait()
            d[k - 1].wait()
        g[k] = pltpu.async_copy(table_h.at[gidxb[b]], msgb[b], semG)
        if k > 0:
            pltpu.sync_copy(msgb[1 - b], acc.at[dstb[1 - b]], add=True)
        d[k] = fire_dst(k, b)
        if has_x:
            if k + 1 < NCH:
                for cp in lin[k + 1]:
                    cp.wait()
                xg[k + 1] = fire_xg(1 - b)
            if k + 2 < NCH:
                lin[k + 2] = fire_lin(k + 2, b)
        else:
            if k + 1 < NCH:
                lin[k + 1] = fire_lin(k + 1, 1 - b)
    bl = (NCH - 1) & 1
    g[NCH - 1].wait()
    d[NCH - 1].wait()
    pltpu.sync_copy(msgb[bl], acc.at[dstb[bl]], add=True)


def _sc_body_l1(src_h, dst_h, et_h, x1d_h, a1_h, r1_h, zer_h, out_h,
                acc, srcb0, srcb1, etb0, etb1, dstb0, dstb1, gidxb0, gidxb1,
                msgb0, msgb1, xsrcb0, xsrcb1, xinit, initb,
                semL, semX, semG, semD):
    c = lax.axis_index("c")
    s = lax.axis_index("s")
    rbase = s * ROWS_PT

    @pl.when(c == 0)
    def _():
        for j in range(5):
            pltpu.sync_copy(x1d_h.at[pl.ds(rbase + j * 128, 128)],
                            xinit.at[j])
        for j in range(5):
            pltpu.async_copy(r1_h.at[xinit.at[j]],
                             initb.at[pl.ds(j * 128, 128)], semG).wait()
        pltpu.sync_copy(initb, acc.at[pl.ds(rbase, ROWS_PT)])

    @pl.when(c == 1)
    def _():
        pltpu.sync_copy(zer_h.at[pl.ds(rbase, ROWS_PT)],
                        acc.at[pl.ds(rbase, ROWS_PT)])

    plsc.subcore_barrier()
    eb = (c * 16 + s) * EPT
    _edge_pipeline(a1_h, src_h, dst_h, et_h, acc, eb,
                   ([srcb0, srcb1], [etb0, etb1], [dstb0, dstb1],
                    [gidxb0, gidxb1], [msgb0, msgb1], [xsrcb0, xsrcb1]),
                   (semL, semX, semG, semD),
                   lambda tv, xv: tv * ATP + xv, x1d_h=x1d_h)
    plsc.subcore_barrier()
    pltpu.sync_copy(acc.at[pl.ds(rbase, ROWS_PT)],
                    out_h.at[c, pl.ds(rbase, ROWS_PT)])


def _sc_body_l2(src_h, dst_h, et_h, y2_h, r2_h, zer_h, out_h,
                acc, srcb0, srcb1, etb0, etb1, dstb0, dstb1, gidxb0, gidxb1,
                msgb0, msgb1,
                semL, semX, semG, semD):
    c = lax.axis_index("c")
    s = lax.axis_index("s")
    rbase = s * ROWS_PT

    @pl.when(c == 0)
    def _():
        pltpu.sync_copy(r2_h.at[pl.ds(rbase, ROWS_PT)],
                        acc.at[pl.ds(rbase, ROWS_PT)])

    @pl.when(c == 1)
    def _():
        pltpu.sync_copy(zer_h.at[pl.ds(rbase, ROWS_PT)],
                        acc.at[pl.ds(rbase, ROWS_PT)])

    plsc.subcore_barrier()
    eb = (c * 16 + s) * EPT
    _edge_pipeline(y2_h, src_h, dst_h, et_h, acc, eb,
                   ([srcb0, srcb1], [etb0, etb1], [dstb0, dstb1],
                    [gidxb0, gidxb1], [msgb0, msgb1], None),
                   (semL, semX, semG, semD),
                   lambda tv, sv: tv * N + sv)
    plsc.subcore_barrier()
    pltpu.sync_copy(acc.at[pl.ds(rbase, ROWS_PT)],
                    out_h.at[c, pl.ds(rbase, ROWS_PT)])


def _idx_bufs():
    per = [pltpu.VMEM((CHUNK,), I32) for _ in range(8)]
    msg = [pltpu.VMEM((CHUNK, 16), F32) for _ in range(2)]
    return per + msg


def _run_sc_l1(src1, dstp, etp, xpad1d, a1f, r1, zer):
    k = pl.kernel(
        _sc_body_l1,
        out_type=jax.ShapeDtypeStruct((2, NPAD, 16), F32),
        mesh=_mesh(),
        compiler_params=pltpu.CompilerParams(use_tc_tiling_on_sc=False),
        scratch_types=[pltpu.VMEM_SHARED((NPAD, 16), F32)]
        + _idx_bufs()
        + [
            pltpu.VMEM((CHUNK,), I32),
            pltpu.VMEM((CHUNK,), I32),
            pltpu.VMEM((5, 128), I32),
            pltpu.VMEM((ROWS_PT, 16), F32),
            pltpu.SemaphoreType.DMA,
            pltpu.SemaphoreType.DMA,
            pltpu.SemaphoreType.DMA,
            pltpu.SemaphoreType.DMA,
        ],
    )
    return k(src1, dstp, etp, xpad1d, a1f, r1, zer)


def _run_sc_l2(src2, dstp, etp, y2f, r2pad, zer):
    k = pl.kernel(
        _sc_body_l2,
        out_type=jax.ShapeDtypeStruct((2, NPAD, 16), F32),
        mesh=_mesh(),
        compiler_params=pltpu.CompilerParams(use_tc_tiling_on_sc=False),
        scratch_types=[pltpu.VMEM_SHARED((NPAD, 16), F32)]
        + _idx_bufs()
        + [
            pltpu.SemaphoreType.DMA,
            pltpu.SemaphoreType.DMA,
            pltpu.SemaphoreType.DMA,
            pltpu.SemaphoreType.DMA,
        ],
    )
    return k(src2, dstp, etp, y2f, r2pad, zer)


def kernel(x, edge_index, edge_attr, atom_emb, edge_emb,
           lin1_W, lin1_b, root1, bias1,
           lin2_W, lin2_b, root2, bias2):
    xs = x[:, 0].astype(I32)
    src = edge_index[0].astype(I32)
    dst = edge_index[1].astype(I32)
    et = edge_attr[:, 0].astype(I32)

    xpad = jnp.concatenate([xs, jnp.full((NPAD - N,), 101, I32)])
    pe = EPAD - E
    src1 = jnp.concatenate([src, jnp.full((pe,), N, I32)])
    src2 = jnp.concatenate([src, jnp.full((pe,), ET * N, I32)])
    dstp = jnp.concatenate([dst, jnp.zeros((pe,), I32)])
    etp = jnp.concatenate([et, jnp.zeros((pe,), I32)])

    atom_pad = jnp.concatenate([atom_emb, jnp.zeros((ATP - 101, 16), F32)])
    w1r = lin1_W.reshape(16, 16, 16)
    b1r = lin1_b.reshape(16, 16)
    w2r = lin2_W.reshape(16, 16, 16)
    b2r = lin2_b.reshape(16, 16)

    a1, r1, w2cat = _run_stage_a(atom_pad, edge_emb, w1r, b1r, root1, bias1,
                                 w2r, b2r)
    a1f = a1.reshape(ET * ATP, 16)

    zer = jnp.zeros((NPAD, 16), F32)
    p = _run_sc_l1(src1, dstp, etp, xpad, a1f, r1, zer)

    y2, r2 = _run_stage_b(p[0, :N], p[1, :N], w2cat, root2, bias2)
    y2f = jnp.concatenate([y2.reshape(ET * N, 16), jnp.zeros((32, 16), F32)])
    r2pad = jnp.concatenate([r2, jnp.zeros((NPAD - N, 16), F32)])

    q = _run_sc_l2(src2, dstp, etp, y2f, r2pad, zer)

    return _run_stage_c(q[0, :N], q[1, :N])


# speedup vs baseline: 17.5953x; 1.5388x over previous
import functools

import jax
import jax.numpy as jnp
from jax import lax
from jax.experimental import pallas as pl
from jax.experimental.pallas import tpu as pltpu
from jax.experimental.pallas import tpu_sc as plsc

N = 10000
E = 320000
ET = 22
ATP = 104
NPAD = 10240
EPAD = 327680
EPT = EPAD // 32
CHUNK = 2048
NCH = EPT // CHUNK
ROWS_PT = NPAD // 16
F32 = jnp.float32
I32 = jnp.int32


def _stage_a(atom_ref, ee_ref, w1r_ref, b1r_ref, root1_ref, b1v_ref,
             w2r_ref, b2r_ref, a1_ref, r1_ref, w2cat_ref):
    at = atom_ref[...]
    w1cat = jnp.concatenate(
        [b1r_ref[...]
         + sum(ee_ref[t:t + 1, e:e + 1] * w1r_ref[e] for e in range(16))
         for t in range(ET)], axis=1)
    w2cat = jnp.concatenate(
        [b2r_ref[...]
         + sum(ee_ref[t:t + 1, e:e + 1] * w2r_ref[e] for e in range(16))
         for t in range(ET)], axis=1)
    w2cat_ref[...] = w2cat
    a1_ref[...] = jnp.dot(at, w1cat, preferred_element_type=F32)
    r1_ref[...] = jnp.dot(at, root1_ref[...],
                          preferred_element_type=F32) + b1v_ref[...]


def _run_stage_a(atom_pad, edge_emb, w1r, b1r, root1, bias1, w2r, b2r):
    return pl.pallas_call(
        _stage_a,
        out_shape=[
            jax.ShapeDtypeStruct((ATP, ET * 16), F32),
            jax.ShapeDtypeStruct((ATP, 16), F32),
            jax.ShapeDtypeStruct((16, ET * 16), F32),
        ],
    )(atom_pad, edge_emb, w1r, b1r, root1, bias1.reshape(1, 16), w2r, b2r)


def _stage_b(p0_ref, p1_ref, w2cat_ref, root2_ref, b2v_ref, y2_ref, r2_ref):
    h = jnp.maximum(p0_ref[...] + p1_ref[...], 0.0)
    y2_ref[...] = jnp.dot(h, w2cat_ref[...], preferred_element_type=F32)
    r2_ref[...] = jnp.dot(h, root2_ref[...],
                          preferred_element_type=F32) + b2v_ref[...]


def _run_stage_b(p0, p1, w2cat, root2, bias2):
    nb = 2000
    return pl.pallas_call(
        _stage_b,
        grid=(N // nb,),
        in_specs=[
            pl.BlockSpec((nb, 16), lambda j: (j, 0)),
            pl.BlockSpec((nb, 16), lambda j: (j, 0)),
            pl.BlockSpec((16, ET * 16), lambda j: (0, 0)),
            pl.BlockSpec((16, 16), lambda j: (0, 0)),
            pl.BlockSpec((1, 16), lambda j: (0, 0)),
        ],
        out_specs=[
            pl.BlockSpec((nb, ET * 16), lambda j: (j, 0)),
            pl.BlockSpec((nb, 16), lambda j: (j, 0)),
        ],
        out_shape=[
            jax.ShapeDtypeStruct((N, ET * 16), F32),
            jax.ShapeDtypeStruct((N, 16), F32),
        ],
    )(p0, p1, w2cat, root2, bias2.reshape(1, 16))


def _stage_c(q0_ref, q1_ref, out_ref):
    out_ref[...] = q0_ref[...] + q1_ref[...]


def _run_stage_c(q0, q1):
    return pl.pallas_call(
        _stage_c,
        out_shape=jax.ShapeDtypeStruct((N, 16), F32),
    )(q0, q1)


@functools.cache
def _mesh():
    return plsc.VectorSubcoreMesh(core_axis_name="c", subcore_axis_name="s",
                                  num_cores=2, num_subcores=16)


def _edge_pipeline(table_h, src_h, dst_h, et_h, acc, eb, bufs, sems,
                   gidx_of, x1d_h=None):
    srcb, etb, dstb, gidxb, msgb, xsrcb = bufs
    semL, semX, semG, semD = sems
    has_x = x1d_h is not None

    def fire_lin(k, b):
        off = eb + k * CHUNK
        return [pltpu.async_copy(src_h.at[pl.ds(off, CHUNK)], srcb[b], semL),
                pltpu.async_copy(et_h.at[pl.ds(off, CHUNK)], etb[b], semL)]

    def fire_dst(k, b):
        off = eb + k * CHUNK
        return pltpu.async_copy(dst_h.at[pl.ds(off, CHUNK)], dstb[b], semD)

    def fire_xg(b):
        return pltpu.async_copy(x1d_h.at[srcb[b]], xsrcb[b], semX)

    def compute(b):
        for i in range(CHUNK // 16):
            sl = pl.ds(i * 16, 16)
            tv = etb[b][sl]
            sv = xsrcb[b][sl] if has_x else srcb[b][sl]
            gidxb[b][sl] = gidx_of(tv, sv)

    lin = {0: fire_lin(0, 0)}
    xg = {}
    g = {}
    d = {}
    if has_x:
        for cp in lin[0]:
            cp.wait()
        xg[0] = fire_xg(0)
        lin[1] = fire_lin(1, 1)
    for k in range(NCH):
        b = k & 1
        if has_x:
            xg[k].wait()
        else:
            for cp in lin[k]:
                cp.wait()
        compute(b)
        if k > 0:
            g[k - 1].wait()
            d[k - 1].wait()
        g[k] = pltpu.async_copy(table_h.at[gidxb[b]], msgb[b], semG)
        if k > 0:
            pltpu.sync_copy(msgb[1 - b], acc.at[dstb[1 - b]], add=True)
        d[k] = fire_dst(k, b)
        if has_x:
            if k + 1 < NCH:
                for cp in lin[k + 1]:
                    cp.wait()
                xg[k + 1] = fire_xg(1 - b)
            if k + 2 < NCH:
                lin[k + 2] = fire_lin(k + 2, b)
        else:
            if k + 1 < NCH:
                lin[k + 1] = fire_lin(k + 1, 1 - b)
    bl = (NCH - 1) & 1
    g[NCH - 1].wait()
    d[NCH - 1].wait()
    pltpu.sync_copy(msgb[bl], acc.at[dstb[bl]], add=True)


def _sc_body_l1(src_h, dst_h, et_h, x1d_h, a1_h, r1_h, zer_h, out_h,
                acc, srcb0, srcb1, etb0, etb1, dstb0, dstb1, gidxb0, gidxb1,
                msgb0, msgb1, xsrcb0, xsrcb1, xinit, initb,
                semL, semX, semG, semD):
    c = lax.axis_index("c")
    s = lax.axis_index("s")
    rbase = s * ROWS_PT

    @pl.when(c == 0)
    def _():
        pltpu.sync_copy(x1d_h.at[pl.ds(rbase, ROWS_PT)], xinit)
        pltpu.async_copy(r1_h.at[xinit], initb, semG).wait()
        pltpu.sync_copy(initb, acc.at[pl.ds(rbase, ROWS_PT)])

    @pl.when(c == 1)
    def _():
        pltpu.sync_copy(zer_h.at[pl.ds(rbase, ROWS_PT)],
                        acc.at[pl.ds(rbase, ROWS_PT)])

    plsc.subcore_barrier()
    eb = (c * 16 + s) * EPT
    _edge_pipeline(a1_h, src_h, dst_h, et_h, acc, eb,
                   ([srcb0, srcb1], [etb0, etb1], [dstb0, dstb1],
                    [gidxb0, gidxb1], [msgb0, msgb1], [xsrcb0, xsrcb1]),
                   (semL, semX, semG, semD),
                   lambda tv, xv: xv * ET + tv, x1d_h=x1d_h)
    plsc.subcore_barrier()
    pltpu.sync_copy(acc.at[pl.ds(rbase, ROWS_PT)],
                    out_h.at[c, pl.ds(rbase, ROWS_PT)])


def _sc_body_l2(src_h, dst_h, et_h, y2_h, r2_h, zer_h, out_h,
                acc, srcb0, srcb1, etb0, etb1, dstb0, dstb1, gidxb0, gidxb1,
                msgb0, msgb1,
                semL, semX, semG, semD):
    c = lax.axis_index("c")
    s = lax.axis_index("s")
    rbase = s * ROWS_PT

    @pl.when(c == 0)
    def _():
        pltpu.sync_copy(r2_h.at[pl.ds(rbase, ROWS_PT)],
                        acc.at[pl.ds(rbase, ROWS_PT)])

    @pl.when(c == 1)
    def _():
        pltpu.sync_copy(zer_h.at[pl.ds(rbase, ROWS_PT)],
                        acc.at[pl.ds(rbase, ROWS_PT)])

    plsc.subcore_barrier()
    eb = (c * 16 + s) * EPT
    _edge_pipeline(y2_h, src_h, dst_h, et_h, acc, eb,
                   ([srcb0, srcb1], [etb0, etb1], [dstb0, dstb1],
                    [gidxb0, gidxb1], [msgb0, msgb1], None),
                   (semL, semX, semG, semD),
                   lambda tv, sv: sv * ET + tv)
    plsc.subcore_barrier()
    pltpu.sync_copy(acc.at[pl.ds(rbase, ROWS_PT)],
                    out_h.at[c, pl.ds(rbase, ROWS_PT)])


def _idx_bufs():
    per = [pltpu.VMEM((CHUNK,), I32) for _ in range(8)]
    msg = [pltpu.VMEM((CHUNK, 16), F32) for _ in range(2)]
    return per + msg


def _run_sc_l1(src1, dstp, etp, xpad1d, a1f, r1, zer):
    k = pl.kernel(
        _sc_body_l1,
        out_type=jax.ShapeDtypeStruct((2, NPAD, 16), F32),
        mesh=_mesh(),
        compiler_params=pltpu.CompilerParams(use_tc_tiling_on_sc=False),
        scratch_types=[pltpu.VMEM_SHARED((NPAD, 16), F32)]
        + _idx_bufs()
        + [
            pltpu.VMEM((CHUNK,), I32),
            pltpu.VMEM((CHUNK,), I32),
            pltpu.VMEM((ROWS_PT,), I32),
            pltpu.VMEM((ROWS_PT, 16), F32),
            pltpu.SemaphoreType.DMA,
            pltpu.SemaphoreType.DMA,
            pltpu.SemaphoreType.DMA,
            pltpu.SemaphoreType.DMA,
        ],
    )
    return k(src1, dstp, etp, xpad1d, a1f, r1, zer)


def _run_sc_l2(src2, dstp, etp, y2f, r2pad, zer):
    k = pl.kernel(
        _sc_body_l2,
        out_type=jax.ShapeDtypeStruct((2, NPAD, 16), F32),
        mesh=_mesh(),
        compiler_params=pltpu.CompilerParams(use_tc_tiling_on_sc=False),
        scratch_types=[pltpu.VMEM_SHARED((NPAD, 16), F32)]
        + _idx_bufs()
        + [
            pltpu.SemaphoreType.DMA,
            pltpu.SemaphoreType.DMA,
            pltpu.SemaphoreType.DMA,
            pltpu.SemaphoreType.DMA,
        ],
    )
    return k(src2, dstp, etp, y2f, r2pad, zer)


def kernel(x, edge_index, edge_attr, atom_emb, edge_emb,
           lin1_W, lin1_b, root1, bias1,
           lin2_W, lin2_b, root2, bias2):
    xs = x[:, 0].astype(I32)
    src = edge_index[0].astype(I32)
    dst = edge_index[1].astype(I32)
    et = edge_attr[:, 0].astype(I32)

    xpad = jnp.concatenate([xs, jnp.full((NPAD - N,), 101, I32)])
    pe = EPAD - E
    src1 = jnp.concatenate([src, jnp.full((pe,), N, I32)])
    src2 = jnp.concatenate([src, jnp.zeros((pe,), I32)])
    dstp = jnp.concatenate([dst, jnp.full((pe,), N, I32)])
    etp = jnp.concatenate([et, jnp.zeros((pe,), I32)])

    atom_pad = jnp.concatenate([atom_emb, jnp.zeros((ATP - 101, 16), F32)])
    w1r = lin1_W.reshape(16, 16, 16)
    b1r = lin1_b.reshape(16, 16)
    w2r = lin2_W.reshape(16, 16, 16)
    b2r = lin2_b.reshape(16, 16)

    a1, r1, w2cat = _run_stage_a(atom_pad, edge_emb, w1r, b1r, root1, bias1,
                                 w2r, b2r)
    a1f = a1.reshape(ATP * ET, 16)

    zer = jnp.zeros((NPAD, 16), F32)
    p = _run_sc_l1(src1, dstp, etp, xpad, a1f, r1, zer)

    y2, r2 = _run_stage_b(p[0, :N], p[1, :N], w2cat, root2, bias2)
    y2f = y2.reshape(N * ET, 16)
    r2pad = jnp.concatenate([r2, jnp.zeros((NPAD - N, 16), F32)])

    q = _run_sc_l2(src2, dstp, etp, y2f, r2pad, zer)

    return _run_stage_c(q[0, :N], q[1, :N])


# speedup vs baseline: 23.7686x; 1.3509x over previous
import functools

import jax
import jax.numpy as jnp
from jax import lax
from jax.experimental import pallas as pl
from jax.experimental.pallas import tpu as pltpu
from jax.experimental.pallas import tpu_sc as plsc

N = 10000
E = 320000
ET = 22
ATP = 104
NPAD = 10240
EPT = E // 32
CHUNK = 2000
NCH = EPT // CHUNK
ROWS_PT = NPAD // 16
F32 = jnp.float32
I32 = jnp.int32


def _stage_a(atom_ref, ee_ref, w1r_ref, b1r_ref, root1_ref, b1v_ref,
             w2r_ref, b2r_ref, a1_ref, r1_ref, w2cat_ref):
    at = atom_ref[...]
    w1cat = jnp.concatenate(
        [b1r_ref[...]
         + sum(ee_ref[t:t + 1, e:e + 1] * w1r_ref[e] for e in range(16))
         for t in range(ET)], axis=1)
    w2cat = jnp.concatenate(
        [b2r_ref[...]
         + sum(ee_ref[t:t + 1, e:e + 1] * w2r_ref[e] for e in range(16))
         for t in range(ET)], axis=1)
    w2cat_ref[...] = w2cat
    a1_ref[...] = jnp.dot(at, w1cat, preferred_element_type=F32)
    r1_ref[...] = jnp.dot(at, root1_ref[...],
                          preferred_element_type=F32) + b1v_ref[...]


def _run_stage_a(atom_pad, edge_emb, w1r, b1r, root1, bias1, w2r, b2r):
    return pl.pallas_call(
        _stage_a,
        out_shape=[
            jax.ShapeDtypeStruct((ATP, ET * 16), F32),
            jax.ShapeDtypeStruct((ATP, 16), F32),
            jax.ShapeDtypeStruct((16, ET * 16), F32),
        ],
    )(atom_pad, edge_emb, w1r, b1r, root1, bias1.reshape(1, 16), w2r, b2r)


def _stage_b(p0_ref, p1_ref, w2cat_ref, root2_ref, b2v_ref, y2_ref, r2_ref):
    h = jnp.maximum(p0_ref[...] + p1_ref[...], 0.0)
    y2_ref[...] = jnp.dot(h, w2cat_ref[...], preferred_element_type=F32)
    r2_ref[...] = jnp.dot(h, root2_ref[...],
                          preferred_element_type=F32) + b2v_ref[...]


def _run_stage_b(p0, p1, w2cat, root2, bias2):
    nb = 2000
    return pl.pallas_call(
        _stage_b,
        grid=(N // nb,),
        in_specs=[
            pl.BlockSpec((nb, 16), lambda j: (j, 0)),
            pl.BlockSpec((nb, 16), lambda j: (j, 0)),
            pl.BlockSpec((16, ET * 16), lambda j: (0, 0)),
            pl.BlockSpec((16, 16), lambda j: (0, 0)),
            pl.BlockSpec((1, 16), lambda j: (0, 0)),
        ],
        out_specs=[
            pl.BlockSpec((nb, ET * 16), lambda j: (j, 0)),
            pl.BlockSpec((nb, 16), lambda j: (j, 0)),
        ],
        out_shape=[
            jax.ShapeDtypeStruct((N, ET * 16), F32),
            jax.ShapeDtypeStruct((N, 16), F32),
        ],
    )(p0, p1, w2cat, root2, bias2.reshape(1, 16))


def _stage_c(q0_ref, q1_ref, out_ref):
    out_ref[...] = q0_ref[...] + q1_ref[...]


def _run_stage_c(q0, q1):
    return pl.pallas_call(
        _stage_c,
        out_shape=jax.ShapeDtypeStruct((N, 16), F32),
    )(q0, q1)


@functools.cache
def _mesh():
    return plsc.VectorSubcoreMesh(core_axis_name="c", subcore_axis_name="s",
                                  num_cores=2, num_subcores=16)


def _edge_pipeline(table_h, ei_h, et_h, acc, eb, bufs, sems,
                   gidx_of, x1d_h=None):
    srcb, etb, dstb, gidxb, msgb, xsrcb = bufs
    semL, semX, semG, semD = sems
    has_x = x1d_h is not None

    def fire_lin(k, b):
        off = eb + k * CHUNK
        return [pltpu.async_copy(ei_h.at[pl.ds(off, CHUNK)], srcb[b], semL),
                pltpu.async_copy(et_h.at[pl.ds(off, CHUNK)], etb[b], semL)]

    def fire_dst(k, b):
        off = eb + k * CHUNK
        return pltpu.async_copy(ei_h.at[pl.ds(E + off, CHUNK)], dstb[b],
                                semD)

    def fire_xg(b):
        return pltpu.async_copy(x1d_h.at[srcb[b]], xsrcb[b], semX)

    def compute(b):
        for i in range(CHUNK // 16):
            sl = pl.ds(i * 16, 16)
            tv = etb[b][sl]
            sv = xsrcb[b][sl] if has_x else srcb[b][sl]
            gidxb[b][sl] = gidx_of(tv, sv)

    lin = {0: fire_lin(0, 0)}
    xg = {}
    g = {}
    d = {}
    if has_x:
        for cp in lin[0]:
            cp.wait()
        xg[0] = fire_xg(0)
        lin[1] = fire_lin(1, 1)
    for k in range(NCH):
        b = k & 1
        if has_x:
            xg[k].wait()
        else:
            for cp in lin[k]:
                cp.wait()
        compute(b)
        if k > 0:
            g[k - 1].wait()
            d[k - 1].wait()
        g[k] = pltpu.async_copy(table_h.at[gidxb[b]], msgb[b], semG)
        if k > 0:
            pltpu.sync_copy(msgb[1 - b], acc.at[dstb[1 - b]], add=True)
        d[k] = fire_dst(k, b)
        if has_x:
            if k + 1 < NCH:
                for cp in lin[k + 1]:
                    cp.wait()
                xg[k + 1] = fire_xg(1 - b)
            if k + 2 < NCH:
                lin[k + 2] = fire_lin(k + 2, b)
        else:
            if k + 1 < NCH:
                lin[k + 1] = fire_lin(k + 1, 1 - b)
    bl = (NCH - 1) & 1
    g[NCH - 1].wait()
    d[NCH - 1].wait()
    pltpu.sync_copy(msgb[bl], acc.at[dstb[bl]], add=True)


def _sc_body_l1(ei_h, et_h, x1d_h, a1_h, r1_h, zer_h, out_h,
                acc, srcb0, srcb1, etb0, etb1, dstb0, dstb1, gidxb0, gidxb1,
                msgb0, msgb1, xsrcb0, xsrcb1, xinit, initb,
                semL, semX, semG, semD):
    c = lax.axis_index("c")
    s = lax.axis_index("s")
    rbase = s * ROWS_PT

    @pl.when(c == 0)
    def _():
        pltpu.sync_copy(x1d_h.at[pl.ds(rbase, ROWS_PT)], xinit)
        pltpu.async_copy(r1_h.at[xinit], initb, semG).wait()
        pltpu.sync_copy(initb, acc.at[pl.ds(rbase, ROWS_PT)])

    @pl.when(c == 1)
    def _():
        pltpu.sync_copy(zer_h.at[pl.ds(rbase, ROWS_PT)],
                        acc.at[pl.ds(rbase, ROWS_PT)])

    plsc.subcore_barrier()
    eb = (c * 16 + s) * EPT
    _edge_pipeline(a1_h, ei_h, et_h, acc, eb,
                   ([srcb0, srcb1], [etb0, etb1], [dstb0, dstb1],
                    [gidxb0, gidxb1], [msgb0, msgb1], [xsrcb0, xsrcb1]),
                   (semL, semX, semG, semD),
                   lambda tv, xv: xv * ET + tv, x1d_h=x1d_h)
    plsc.subcore_barrier()
    pltpu.sync_copy(acc.at[pl.ds(rbase, ROWS_PT)],
                    out_h.at[c, pl.ds(rbase, ROWS_PT)])


def _sc_body_l2(ei_h, et_h, y2_h, r2_h, zer_h, out_h,
                acc, srcb0, srcb1, etb0, etb1, dstb0, dstb1, gidxb0, gidxb1,
                msgb0, msgb1,
                semL, semX, semG, semD):
    c = lax.axis_index("c")
    s = lax.axis_index("s")
    rbase = s * ROWS_PT

    @pl.when(c == 0)
    def _():
        pltpu.sync_copy(r2_h.at[pl.ds(rbase, ROWS_PT)],
                        acc.at[pl.ds(rbase, ROWS_PT)])

    @pl.when(c == 1)
    def _():
        pltpu.sync_copy(zer_h.at[pl.ds(rbase, ROWS_PT)],
                        acc.at[pl.ds(rbase, ROWS_PT)])

    plsc.subcore_barrier()
    eb = (c * 16 + s) * EPT
    _edge_pipeline(y2_h, ei_h, et_h, acc, eb,
                   ([srcb0, srcb1], [etb0, etb1], [dstb0, dstb1],
                    [gidxb0, gidxb1], [msgb0, msgb1], None),
                   (semL, semX, semG, semD),
                   lambda tv, sv: sv * ET + tv)
    plsc.subcore_barrier()
    pltpu.sync_copy(acc.at[pl.ds(rbase, ROWS_PT)],
                    out_h.at[c, pl.ds(rbase, ROWS_PT)])


def _idx_bufs():
    per = [pltpu.VMEM((CHUNK,), I32) for _ in range(8)]
    msg = [pltpu.VMEM((CHUNK, 16), F32) for _ in range(2)]
    return per + msg


def _run_sc_l1(ei, et, xpad1d, a1f, r1, zer):
    k = pl.kernel(
        _sc_body_l1,
        out_type=jax.ShapeDtypeStruct((2, NPAD, 16), F32),
        mesh=_mesh(),
        compiler_params=pltpu.CompilerParams(use_tc_tiling_on_sc=False),
        scratch_types=[pltpu.VMEM_SHARED((NPAD, 16), F32)]
        + _idx_bufs()
        + [
            pltpu.VMEM((CHUNK,), I32),
            pltpu.VMEM((CHUNK,), I32),
            pltpu.VMEM((ROWS_PT,), I32),
            pltpu.VMEM((ROWS_PT, 16), F32),
            pltpu.SemaphoreType.DMA,
            pltpu.SemaphoreType.DMA,
            pltpu.SemaphoreType.DMA,
            pltpu.SemaphoreType.DMA,
        ],
    )
    return k(ei, et, xpad1d, a1f, r1, zer)


def _run_sc_l2(ei, et, y2f, r2pad, zer):
    k = pl.kernel(
        _sc_body_l2,
        out_type=jax.ShapeDtypeStruct((2, NPAD, 16), F32),
        mesh=_mesh(),
        compiler_params=pltpu.CompilerParams(use_tc_tiling_on_sc=False),
        scratch_types=[pltpu.VMEM_SHARED((NPAD, 16), F32)]
        + _idx_bufs()
        + [
            pltpu.SemaphoreType.DMA,
            pltpu.SemaphoreType.DMA,
            pltpu.SemaphoreType.DMA,
            pltpu.SemaphoreType.DMA,
        ],
    )
    return k(ei, et, y2f, r2pad, zer)


def kernel(x, edge_index, edge_attr, atom_emb, edge_emb,
           lin1_W, lin1_b, root1, bias1,
           lin2_W, lin2_b, root2, bias2):
    ei = edge_index.reshape(2 * E).astype(I32)
    et = edge_attr.reshape(E).astype(I32)
    xpad = jnp.concatenate([x.reshape(N).astype(I32),
                            jnp.zeros((NPAD - N,), I32)])

    atom_pad = jnp.concatenate([atom_emb, jnp.zeros((ATP - 101, 16), F32)])
    w1r = lin1_W.reshape(16, 16, 16)
    b1r = lin1_b.reshape(16, 16)
    w2r = lin2_W.reshape(16, 16, 16)
    b2r = lin2_b.reshape(16, 16)

    a1, r1, w2cat = _run_stage_a(atom_pad, edge_emb, w1r, b1r, root1, bias1,
                                 w2r, b2r)
    a1f = a1.reshape(ATP * ET, 16)

    zer = jnp.zeros((NPAD, 16), F32)
    p = _run_sc_l1(ei, et, xpad, a1f, r1, zer)

    y2, r2 = _run_stage_b(p[0, :N], p[1, :N], w2cat, root2, bias2)
    y2f = y2.reshape(N * ET, 16)
    r2pad = jnp.concatenate([r2, jnp.zeros((NPAD - N, 16), F32)])

    q = _run_sc_l2(ei, et, y2f, r2pad, zer)

    return _run_stage_c(q[0, :N], q[1, :N])


# speedup vs baseline: 29.9787x; 1.2613x over previous
import functools

import jax
import jax.numpy as jnp
from jax import lax
from jax.experimental import pallas as pl
from jax.experimental.pallas import tpu as pltpu
from jax.experimental.pallas import tpu_sc as plsc

N = 10000
E = 320000
ET = 22
ATP = 104
NPAD = 10240
EPT = E // 32
CHUNK = 2000
NCH = EPT // CHUNK
ROWS_PT = NPAD // 16
F32 = jnp.float32
I32 = jnp.int32


def _stage_a(atom_ref, ee_ref, w1r_ref, b1r_ref, root1_ref, b1v_ref,
             w2r_ref, b2r_ref, a1_ref, r1_ref, w2cat_ref):
    at = atom_ref[...]
    w1cat = jnp.concatenate(
        [b1r_ref[...]
         + sum(ee_ref[t:t + 1, e:e + 1] * w1r_ref[e] for e in range(16))
         for t in range(ET)], axis=1)
    w2cat = jnp.concatenate(
        [b2r_ref[...]
         + sum(ee_ref[t:t + 1, e:e + 1] * w2r_ref[e] for e in range(16))
         for t in range(ET)], axis=1)
    w2cat_ref[...] = w2cat
    a1_ref[...] = jnp.dot(at, w1cat, preferred_element_type=F32)
    r1_ref[...] = jnp.dot(at, root1_ref[...],
                          preferred_element_type=F32) + b1v_ref[...]


def _run_stage_a(atom_pad, edge_emb, w1r, b1r, root1, bias1, w2r, b2r):
    return pl.pallas_call(
        _stage_a,
        out_shape=[
            jax.ShapeDtypeStruct((ATP, ET * 16), F32),
            jax.ShapeDtypeStruct((ATP, 16), F32),
            jax.ShapeDtypeStruct((16, ET * 16), F32),
        ],
    )(atom_pad, edge_emb, w1r, b1r, root1, bias1.reshape(1, 16), w2r, b2r)


def _stage_b(p_ref, w2cat_ref, root2_ref, b2v_ref, y2_ref, r2_ref):
    h = jnp.maximum(p_ref[0] + p_ref[1], 0.0)
    y2_ref[...] = jnp.dot(h, w2cat_ref[...], preferred_element_type=F32)
    r2_ref[...] = jnp.dot(h, root2_ref[...],
                          preferred_element_type=F32) + b2v_ref[...]


def _run_stage_b(p, w2cat, root2, bias2):
    nb = 2000
    return pl.pallas_call(
        _stage_b,
        grid=(N // nb,),
        in_specs=[
            pl.BlockSpec((2, nb, 16), lambda j: (0, j, 0)),
            pl.BlockSpec((16, ET * 16), lambda j: (0, 0)),
            pl.BlockSpec((16, 16), lambda j: (0, 0)),
            pl.BlockSpec((1, 16), lambda j: (0, 0)),
        ],
        out_specs=[
            pl.BlockSpec((nb, ET * 16), lambda j: (j, 0)),
            pl.BlockSpec((nb, 16), lambda j: (j, 0)),
        ],
        out_shape=[
            jax.ShapeDtypeStruct((N, ET * 16), F32),
            jax.ShapeDtypeStruct((N, 16), F32),
        ],
    )(p, w2cat, root2, bias2.reshape(1, 16))


def _stage_c(q_ref, out_ref):
    out_ref[...] = q_ref[0] + q_ref[1]


def _run_stage_c(q):
    nb = 2000
    return pl.pallas_call(
        _stage_c,
        grid=(N // nb,),
        in_specs=[pl.BlockSpec((2, nb, 16), lambda j: (0, j, 0))],
        out_specs=pl.BlockSpec((nb, 16), lambda j: (j, 0)),
        out_shape=jax.ShapeDtypeStruct((N, 16), F32),
    )(q)


@functools.cache
def _mesh():
    return plsc.VectorSubcoreMesh(core_axis_name="c", subcore_axis_name="s",
                                  num_cores=2, num_subcores=16)


def _edge_pipeline(table_h, ei_h, et_h, acc, eb, bufs, sems,
                   gidx_of, x1d_h=None):
    srcb, etb, dstb, gidxb, msgb, xtab = bufs
    semL, semX, semG, semD = sems
    has_x = xtab is not None

    def fire_lin(k, b):
        off = eb + k * CHUNK
        return [pltpu.async_copy(ei_h.at[pl.ds(off, CHUNK)], srcb[b], semL),
                pltpu.async_copy(et_h.at[pl.ds(off, CHUNK)], etb[b],
                                 semL)]

    def fire_dst(k, b):
        off = eb + k * CHUNK
        return pltpu.async_copy(ei_h.at[pl.ds(E + off, CHUNK)], dstb[b],
                                semD)

    def compute(b):
        for i in range(CHUNK // 16):
            sl = pl.ds(i * 16, 16)
            tv = etb[b][sl]
            sv = srcb[b][sl]
            if has_x:
                sv = plsc.load_gather(xtab, [sv])
            gidxb[b][sl] = gidx_of(tv, sv)

    lin = {0: fire_lin(0, 0)}
    g = {}
    d = {}
    for k in range(NCH):
        b = k & 1
        for cp in lin[k]:
            cp.wait()
        compute(b)
        if k > 0:
            g[k - 1].wait()
            d[k - 1].wait()
        g[k] = pltpu.async_copy(table_h.at[gidxb[b]], msgb[b], semG)
        if k > 0:
            pltpu.sync_copy(msgb[1 - b], acc.at[dstb[1 - b]], add=True)
        d[k] = fire_dst(k, b)
        if k + 1 < NCH:
            lin[k + 1] = fire_lin(k + 1, 1 - b)
    bl = (NCH - 1) & 1
    g[NCH - 1].wait()
    d[NCH - 1].wait()
    pltpu.sync_copy(msgb[bl], acc.at[dstb[bl]], add=True)


def _sc_body_l1(ei_h, et_h, x1d_h, a1_h, r1_h, zer_h, out_h,
                acc, srcb0, srcb1, etb0, etb1, dstb0, dstb1, gidxb0, gidxb1,
                msgb0, msgb1, xtab, xinit, initb,
                semL, semX, semG, semD):
    c = lax.axis_index("c")
    s = lax.axis_index("s")
    rbase = s * ROWS_PT
    pltpu.sync_copy(x1d_h, xtab)

    @pl.when(c == 0)
    def _():
        pltpu.sync_copy(x1d_h.at[pl.ds(rbase, ROWS_PT)], xinit)
        pltpu.async_copy(r1_h.at[xinit], initb, semG).wait()
        pltpu.sync_copy(initb, acc.at[pl.ds(rbase, ROWS_PT)])

    @pl.when(c == 1)
    def _():
        pltpu.sync_copy(zer_h.at[pl.ds(rbase, ROWS_PT)],
                        acc.at[pl.ds(rbase, ROWS_PT)])

    plsc.subcore_barrier()
    eb = (c * 16 + s) * EPT
    _edge_pipeline(a1_h, ei_h, et_h, acc, eb,
                   ([srcb0, srcb1], [etb0, etb1], [dstb0, dstb1],
                    [gidxb0, gidxb1], [msgb0, msgb1], xtab),
                   (semL, semX, semG, semD),
                   lambda tv, xv: xv * ET + tv, x1d_h=x1d_h)
    plsc.subcore_barrier()
    pltpu.sync_copy(acc.at[pl.ds(rbase, ROWS_PT)],
                    out_h.at[c, pl.ds(rbase, ROWS_PT)])


def _sc_body_l2(ei_h, et_h, y2_h, r2_h, zer_h, out_h,
                acc, srcb0, srcb1, etb0, etb1, dstb0, dstb1, gidxb0, gidxb1,
                msgb0, msgb1,
                semL, semX, semG, semD):
    c = lax.axis_index("c")
    s = lax.axis_index("s")
    rbase = s * ROWS_PT

    @pl.when(c == 0)
    def _():
        pltpu.sync_copy(r2_h.at[pl.ds(rbase, ROWS_PT)],
                        acc.at[pl.ds(rbase, ROWS_PT)])

    @pl.when(c == 1)
    def _():
        pltpu.sync_copy(zer_h.at[pl.ds(rbase, ROWS_PT)],
                        acc.at[pl.ds(rbase, ROWS_PT)])

    plsc.subcore_barrier()
    eb = (c * 16 + s) * EPT
    _edge_pipeline(y2_h, ei_h, et_h, acc, eb,
                   ([srcb0, srcb1], [etb0, etb1], [dstb0, dstb1],
                    [gidxb0, gidxb1], [msgb0, msgb1], None),
                   (semL, semX, semG, semD),
                   lambda tv, sv: sv * ET + tv)
    plsc.subcore_barrier()
    pltpu.sync_copy(acc.at[pl.ds(rbase, ROWS_PT)],
                    out_h.at[c, pl.ds(rbase, ROWS_PT)])


def _idx_bufs():
    per = [pltpu.VMEM((CHUNK,), I32) for _ in range(8)]
    msg = [pltpu.VMEM((CHUNK, 16), F32) for _ in range(2)]
    return per + msg


def _run_sc_l1(ei, et, xpad1d, a1f, r1, zer):
    k = pl.kernel(
        _sc_body_l1,
        out_type=jax.ShapeDtypeStruct((2, NPAD, 16), F32),
        mesh=_mesh(),
        compiler_params=pltpu.CompilerParams(use_tc_tiling_on_sc=False,
                                             needs_layout_passes=False),
        scratch_types=[pltpu.VMEM_SHARED((NPAD, 16), F32)]
        + _idx_bufs()
        + [
            pltpu.VMEM((NPAD,), I32),
            pltpu.VMEM((ROWS_PT,), I32),
            pltpu.VMEM((ROWS_PT, 16), F32),
            pltpu.SemaphoreType.DMA,
            pltpu.SemaphoreType.DMA,
            pltpu.SemaphoreType.DMA,
            pltpu.SemaphoreType.DMA,
        ],
    )
    return k(ei, et, xpad1d, a1f, r1, zer)


def _run_sc_l2(ei, et, y2f, r2pad, zer):
    k = pl.kernel(
        _sc_body_l2,
        out_type=jax.ShapeDtypeStruct((2, NPAD, 16), F32),
        mesh=_mesh(),
        compiler_params=pltpu.CompilerParams(use_tc_tiling_on_sc=False),
        scratch_types=[pltpu.VMEM_SHARED((NPAD, 16), F32)]
        + _idx_bufs()
        + [
            pltpu.SemaphoreType.DMA,
            pltpu.SemaphoreType.DMA,
            pltpu.SemaphoreType.DMA,
            pltpu.SemaphoreType.DMA,
        ],
    )
    return k(ei, et, y2f, r2pad, zer)


def kernel(x, edge_index, edge_attr, atom_emb, edge_emb,
           lin1_W, lin1_b, root1, bias1,
           lin2_W, lin2_b, root2, bias2):
    ei = edge_index.reshape(2 * E).astype(I32)
    et = edge_attr.reshape(E).astype(I32)
    xpad = jnp.concatenate([x.reshape(N).astype(I32),
                            jnp.zeros((NPAD - N,), I32)])

    atom_pad = jnp.concatenate([atom_emb, jnp.zeros((ATP - 101, 16), F32)])
    w1r = lin1_W.reshape(16, 16, 16)
    b1r = lin1_b.reshape(16, 16)
    w2r = lin2_W.reshape(16, 16, 16)
    b2r = lin2_b.reshape(16, 16)

    a1, r1, w2cat = _run_stage_a(atom_pad, edge_emb, w1r, b1r, root1, bias1,
                                 w2r, b2r)
    a1f = a1.reshape(ATP * ET, 16)

    zer = jnp.zeros((NPAD, 16), F32)
    p = _run_sc_l1(ei, et, xpad, a1f, r1, zer)

    y2, r2 = _run_stage_b(p, w2cat, root2, bias2)
    y2f = y2.reshape(N * ET, 16)
    r2pad = jnp.concatenate([r2, jnp.zeros((NPAD - N, 16), F32)])

    q = _run_sc_l2(ei, et, y2f, r2pad, zer)

    return _run_stage_c(q)
